# Initial kernel scaffold; baseline (speedup 1.0000x reference)
import jax
import jax.numpy as jnp
from jax.experimental import pallas as pl


def kernel(input_ids, token_type_ids, word_emb, pos_emb, type_emb, ln_gamma, ln_beta):
    raise NotImplementedError("write your pallas kernel here")


# speedup vs baseline: 5.2459x; 5.2459x over previous
import functools

import jax
import jax.numpy as jnp
from jax import lax
from jax.experimental import pallas as pl
from jax.experimental.pallas import tpu as pltpu
from jax.experimental.pallas import tpu_sc as plsc

VOCAB = 100000
HIDDEN = 128
MAX_POS = 512
B, S = 1024, 200
EPS = 1e-12

NC, NS = 2, 16
NW = NC * NS
TOK = B * S
PER_W = TOK // NW
CH = 128
ITERS = PER_W // CH


def _sc_gather(ids_hbm, word_hbm, out_hbm, idx_v, rows_v, sem):
    wid = lax.axis_index("s") * NC + lax.axis_index("c")
    base = wid * PER_W

    def body(i, carry):
        off = base + i * CH
        pltpu.sync_copy(ids_hbm.at[pl.ds(off, CH)], idx_v)
        pltpu.async_copy(word_hbm.at[idx_v], rows_v, sem).wait()
        pltpu.sync_copy(rows_v, out_hbm.at[pl.ds(off, CH)])
        return carry

    lax.fori_loop(0, ITERS, body, 0)


_sc_gather_call = functools.partial(
    pl.kernel,
    mesh=plsc.VectorSubcoreMesh(core_axis_name="c", subcore_axis_name="s"),
    out_type=jax.ShapeDtypeStruct((TOK, HIDDEN), jnp.float32),
    scratch_types=[
        pltpu.VMEM((CH,), jnp.int32),
        pltpu.VMEM((CH, HIDDEN), jnp.float32),
        pltpu.SemaphoreType.DMA,
    ],
)(_sc_gather)


def _ln_body(g_ref, tt_ref, pos_ref, type_ref, gam_ref, bet_ref, o_ref):
    x = g_ref[...]
    tt = tt_ref[...].astype(jnp.float32)[:, :, None]
    t0 = type_ref[0][None, None, :]
    dt = (type_ref[1] - type_ref[0])[None, None, :]
    x = x + pos_ref[...][None, :, :] + t0 + tt * dt
    mean = jnp.mean(x, axis=-1, keepdims=True)
    xc = x - mean
    var = jnp.mean(xc * xc, axis=-1, keepdims=True)
    o_ref[...] = xc * lax.rsqrt(var + EPS) * gam_ref[...] + bet_ref[...]


GB = 8


def kernel(input_ids, token_type_ids, word_emb, pos_emb, type_emb, ln_gamma, ln_beta):
    ids_flat = input_ids.reshape(TOK).astype(jnp.int32)
    gathered = _sc_gather_call(ids_flat, word_emb)
    gathered = gathered.reshape(B, S, HIDDEN)

    out = pl.pallas_call(
        _ln_body,
        grid=(B // GB,),
        in_specs=[
            pl.BlockSpec((GB, S, HIDDEN), lambda i: (i, 0, 0)),
            pl.BlockSpec((GB, S), lambda i: (i, 0)),
            pl.BlockSpec((S, HIDDEN), lambda i: (0, 0)),
            pl.BlockSpec((2, HIDDEN), lambda i: (0, 0)),
            pl.BlockSpec((HIDDEN,), lambda i: (0,)),
            pl.BlockSpec((HIDDEN,), lambda i: (0,)),
        ],
        out_specs=pl.BlockSpec((GB, S, HIDDEN), lambda i: (i, 0, 0)),
        out_shape=jax.ShapeDtypeStruct((B, S, HIDDEN), jnp.float32),
    )(gathered, token_type_ids.astype(jnp.int32), pos_emb[:S], type_emb,
      ln_gamma, ln_beta)
    return out
